# baseline (speedup 1.0000x reference)
import jax
import jax.numpy as jnp
from jax.experimental import pallas as pl
from jax.experimental.pallas import tpu as pltpu

N = 10000
D = 128
BM = 200


def _leaky(x):
    return jnp.where(x >= 0, x, 0.01 * x)


def _body(a_ref, ego_ref, wgc_ref, bgc_ref, wbi_ref, bbi_ref, out_ref):
    i = pl.program_id(0)
    nb = jnp.dot(a_ref[...], ego_ref[...], preferred_element_type=jnp.float32)
    ego = ego_ref[pl.ds(i * BM, BM), :]
    add = jax.lax.dot_general(ego + nb, wgc_ref[...],
                              (((1,), (1,)), ((), ())),
                              preferred_element_type=jnp.float32)
    wise = jax.lax.dot_general(ego * nb, wbi_ref[...],
                               (((1,), (1,)), ((), ())),
                               preferred_element_type=jnp.float32)
    out_ref[...] = _leaky(add + bgc_ref[...]) + _leaky(wise + bbi_ref[...])


@jax.jit
def kernel(ego_embed, A_in, W_gc, b_gc, W_bi, b_bi):
    return pl.pallas_call(
        _body,
        grid=(N // BM,),
        in_specs=[
            pl.BlockSpec((BM, N), lambda i: (i, 0)),
            pl.BlockSpec((N, D), lambda i: (0, 0)),
            pl.BlockSpec((D, D), lambda i: (0, 0)),
            pl.BlockSpec((1, D), lambda i: (0, 0)),
            pl.BlockSpec((D, D), lambda i: (0, 0)),
            pl.BlockSpec((1, D), lambda i: (0, 0)),
        ],
        out_specs=pl.BlockSpec((BM, D), lambda i: (i, 0)),
        out_shape=jax.ShapeDtypeStruct((N, D), jnp.float32),
        compiler_params=pltpu.CompilerParams(
            dimension_semantics=("parallel",),
        ),
    )(A_in, ego_embed, W_gc, b_gc.reshape(1, D), W_bi, b_bi.reshape(1, D))


# speedup vs baseline: 1.0273x; 1.0273x over previous
import jax
import jax.numpy as jnp
from jax.experimental import pallas as pl
from jax.experimental.pallas import tpu as pltpu

N = 10000
D = 128
BM = 400


def _leaky(x):
    return jnp.where(x >= 0, x, 0.01 * x)


def _body(a_ref, ego_ref, wgc_ref, bgc_ref, wbi_ref, bbi_ref, out_ref):
    i = pl.program_id(0)
    nb = jnp.dot(a_ref[...], ego_ref[...], preferred_element_type=jnp.float32)
    ego = ego_ref[pl.ds(i * BM, BM), :]
    add = jax.lax.dot_general(ego + nb, wgc_ref[...],
                              (((1,), (1,)), ((), ())),
                              preferred_element_type=jnp.float32)
    wise = jax.lax.dot_general(ego * nb, wbi_ref[...],
                               (((1,), (1,)), ((), ())),
                               preferred_element_type=jnp.float32)
    out_ref[...] = _leaky(add + bgc_ref[...]) + _leaky(wise + bbi_ref[...])


@jax.jit
def kernel(ego_embed, A_in, W_gc, b_gc, W_bi, b_bi):
    return pl.pallas_call(
        _body,
        grid=(N // BM,),
        in_specs=[
            pl.BlockSpec((BM, N), lambda i: (i, 0)),
            pl.BlockSpec((N, D), lambda i: (0, 0)),
            pl.BlockSpec((D, D), lambda i: (0, 0)),
            pl.BlockSpec((1, D), lambda i: (0, 0)),
            pl.BlockSpec((D, D), lambda i: (0, 0)),
            pl.BlockSpec((1, D), lambda i: (0, 0)),
        ],
        out_specs=pl.BlockSpec((BM, D), lambda i: (i, 0)),
        out_shape=jax.ShapeDtypeStruct((N, D), jnp.float32),
        compiler_params=pltpu.CompilerParams(
            dimension_semantics=("parallel",),
        ),
    )(A_in, ego_embed, W_gc, b_gc.reshape(1, D), W_bi, b_bi.reshape(1, D))
